# Initial kernel scaffold; baseline (speedup 1.0000x reference)
import jax
import jax.numpy as jnp
from jax.experimental import pallas as pl


def kernel(node_feats, edge_weights, rel_fcs, skip_w, skip_b, edge_index):
    raise NotImplementedError("write your pallas kernel here")


# speedup vs baseline: 3.9538x; 3.9538x over previous
import jax
import jax.numpy as jnp
from jax import lax
from jax.experimental import pallas as pl
from jax.experimental.pallas import tpu as pltpu
from jax.experimental.pallas import tpu_sc as plsc

N = 10000
E = 320000
D = 128
R = 8

NC = 2
NS = 16
NW = NC * NS
L = 16

C = 32
EPT = 10016
EPAD = EPT * NW
NCHUNK = EPT // C
NA = N + 16
ND = 80


_BM = 400


def _mm_body(x_ref, wcat_ref, wskip_ref, b_ref, y_ref, s_ref):
    x = x_ref[...]
    y_ref[...] = jnp.dot(x, wcat_ref[...], preferred_element_type=jnp.float32)
    s_ref[...] = (
        jnp.dot(x, wskip_ref[...], preferred_element_type=jnp.float32)
        + b_ref[...]
    )


def _premultiply(x, wcat, wskip, b2d):
    return pl.pallas_call(
        _mm_body,
        grid=(N // _BM,),
        in_specs=[
            pl.BlockSpec((_BM, D), lambda i: (i, 0)),
            pl.BlockSpec((D, R * D), lambda i: (0, 0)),
            pl.BlockSpec((D, D), lambda i: (0, 0)),
            pl.BlockSpec((1, D), lambda i: (0, 0)),
        ],
        out_specs=[
            pl.BlockSpec((_BM, R * D), lambda i: (i, 0)),
            pl.BlockSpec((_BM, D), lambda i: (i, 0)),
        ],
        out_shape=[
            jax.ShapeDtypeStruct((N, R * D), jnp.float32),
            jax.ShapeDtypeStruct((N, D), jnp.float32),
        ],
    )(x, wcat, wskip, b2d)


def _sc_body(y_hbm, src_hbm, dst_hbm, wt_hbm, z2d_hbm,
             acc_out, degw_out,
             src_v, dst_v, drow_v, w_v, rows_v, m_v, m2_v, acc_sh, deg_sh,
             sem):
    c = lax.axis_index("c")
    s = lax.axis_index("s")
    wid = c * NS + s

    @pl.when(s < 10)
    def _():
        pltpu.sync_copy(z2d_hbm, acc_sh.at[pl.ds(s * 1000, 1000)])

    @pl.when(s == 10)
    def _():
        pltpu.sync_copy(z2d_hbm.at[pl.ds(0, 16)], acc_sh.at[pl.ds(N, 16)])

    @pl.when(s == 11)
    def _():
        pltpu.sync_copy(z2d_hbm.at[pl.ds(0, ND)], deg_sh)

    plsc.subcore_barrier()

    lane = lax.iota(jnp.int32, L)

    def chunk_body(g, _):
        base = wid * EPT + g * C
        pltpu.sync_copy(src_hbm.at[pl.ds(base, C)], src_v)
        pltpu.sync_copy(dst_hbm.at[pl.ds(base, C)], dst_v)
        pltpu.sync_copy(wt_hbm.at[pl.ds(base * R, C * R)], w_v)
        pltpu.async_copy(y_hbm.at[src_v], rows_v, sem).wait()

        for k in range(C // L):
            drow_v[pl.ds(k * L, L)] = lax.shift_right_logical(
                dst_v[pl.ds(k * L, L)], 7
            )

        def blk_body(b, _):
            dst16 = dst_v[pl.ds(b * L, L)]
            dmod = jnp.bitwise_and(dst16, 127)
            ws = [w_v[pl.ds(b * L * R + k * L, L)] for k in range(R)]
            for e in range(L):
                i = b * L + e
                m = [None] * (D // L)
                for r in range(R):
                    wb = jnp.broadcast_to(ws[e // 2][(e % 2) * R + r], (L,))
                    for j in range(D // L):
                        v = rows_v[i, pl.ds(r * D + j * L, L)] * wb
                        m[j] = v if r == 0 else m[j] + v
                for j in range(D // L):
                    m_v[i, pl.ds(j * L, L)] = m[j]
                dm = jnp.broadcast_to(dmod[e], (L,))
                for j in range(D // L):
                    m2_v[i, pl.ds(j * L, L)] = jnp.where(
                        lane + (j * L) == dm, 1.0, 0.0
                    )
            return 0

        lax.fori_loop(0, C // L, blk_body, 0, unroll=False)

        pltpu.sync_copy(m_v, acc_sh.at[dst_v], add=True)
        pltpu.sync_copy(m2_v, deg_sh.at[drow_v], add=True)
        return 0

    lax.fori_loop(0, NCHUNK, chunk_body, 0, unroll=False)

    plsc.subcore_barrier()

    @pl.when(s < 10)
    def _():
        pltpu.sync_copy(acc_sh.at[pl.ds(s * 1000, 1000)],
                        acc_out.at[c, pl.ds(s * 1000, 1000)])

    @pl.when(s == 11)
    def _():
        pltpu.sync_copy(deg_sh, degw_out.at[c])


_sc_edge_pass = pl.kernel(
    _sc_body,
    out_type=[
        jax.ShapeDtypeStruct((NC, N, D), jnp.float32),
        jax.ShapeDtypeStruct((NC, ND, D), jnp.float32),
    ],
    mesh=plsc.VectorSubcoreMesh(
        core_axis_name="c", subcore_axis_name="s", num_cores=NC,
        num_subcores=NS,
    ),
    scratch_types=[
        pltpu.VMEM((C,), jnp.int32),
        pltpu.VMEM((C,), jnp.int32),
        pltpu.VMEM((C,), jnp.int32),
        pltpu.VMEM((C * R,), jnp.float32),
        pltpu.VMEM((C, R * D), jnp.float32),
        pltpu.VMEM((C, D), jnp.float32),
        pltpu.VMEM((C, D), jnp.float32),
        pltpu.VMEM_SHARED((NA, D), jnp.float32),
        pltpu.VMEM_SHARED((ND, D), jnp.float32),
        pltpu.SemaphoreType.DMA,
    ],
)


def _combine_body(a0_ref, a1_ref, d_ref, s_ref, o_ref):
    deg = jnp.maximum(d_ref[...], 1.0)
    o_ref[...] = (a0_ref[...] + a1_ref[...]) / deg + s_ref[...]


def _combine(a0, a1, d, skip):
    return pl.pallas_call(
        _combine_body,
        grid=(N // _BM,),
        in_specs=[
            pl.BlockSpec((_BM, D), lambda i: (i, 0)),
            pl.BlockSpec((_BM, D), lambda i: (i, 0)),
            pl.BlockSpec((_BM, 1), lambda i: (i, 0)),
            pl.BlockSpec((_BM, D), lambda i: (i, 0)),
        ],
        out_specs=pl.BlockSpec((_BM, D), lambda i: (i, 0)),
        out_shape=jax.ShapeDtypeStruct((N, D), jnp.float32),
    )(a0, a1, d, skip)


@jax.jit
def kernel(node_feats, edge_weights, rel_fcs, skip_w, skip_b, edge_index):
    wcat = rel_fcs.transpose(1, 0, 2).reshape(D, R * D)
    y, skip = _premultiply(node_feats, wcat, skip_w, skip_b.reshape(1, D))

    npad = EPAD - E
    src = jnp.concatenate([edge_index[0], jnp.zeros((npad,), jnp.int32)])
    dst = jnp.concatenate([edge_index[1], jnp.full((npad,), N, jnp.int32)])
    wt = jnp.concatenate(
        [edge_weights.T, jnp.zeros((npad, R), jnp.float32)]
    ).reshape(EPAD * R)

    z2d = jnp.zeros((1000, D), jnp.float32)
    acc, degw = _sc_edge_pass(y, src, dst, wt, z2d)

    deg = (degw[0] + degw[1]).reshape(ND * D)[:N].reshape(N, 1)
    return _combine(acc[0], acc[1], deg, skip)
